# baseline (speedup 1.0000x reference)
import functools

import jax
import jax.numpy as jnp
from jax import lax
from jax.experimental import pallas as pl
from jax.experimental.pallas import tpu as pltpu
from jax.experimental.pallas import tpu_sc as plsc

_E = 64
_K = 8
_NEG = -3.0e38


def _logits_block(h_ref, w_ref, out_ref):
    out_ref[...] = lax.dot_general(
        w_ref[...], h_ref[...], (((1,), (1,)), ((), ())),
        preferred_element_type=jnp.float32)


def _tc_logits_chunk(hf, W, c, chunk):
    _, d = hf.shape
    blk = 4096
    g = chunk // blk
    return pl.pallas_call(
        _logits_block,
        grid=(g,),
        in_specs=[
            pl.BlockSpec((blk, d), lambda i, _o=c * g: (_o + i, 0)),
            pl.BlockSpec((_E, d), lambda i: (0, 0)),
        ],
        out_specs=pl.BlockSpec((_E, blk), lambda i: (0, i)),
        out_shape=jax.ShapeDtypeStruct((_E, chunk), jnp.float32),
    )(hf, W)


def _make_sc_topk(n):
    info = plsc.get_sparse_core_info()
    nw = info.num_cores * info.num_subcores
    per_w = n // nw
    groups = per_w // 16
    mesh = plsc.VectorSubcoreMesh(core_axis_name="c", subcore_axis_name="s")

    @functools.partial(
        pl.kernel, mesh=mesh,
        out_type=[jax.ShapeDtypeStruct((_K, n), jnp.int32),
                  jax.ShapeDtypeStruct((_K, n), jnp.float32)],
        scratch_types=[pltpu.VMEM((_E, per_w), jnp.float32),
                       pltpu.VMEM((_K, per_w), jnp.int32),
                       pltpu.VMEM((_K, per_w), jnp.float32)],
    )
    def sc_topk(logits_hbm, ids_hbm, wts_hbm, sc_v, ids_v, wts_v):
        wid = lax.axis_index("s") * info.num_cores + lax.axis_index("c")
        base = wid * per_w
        pltpu.sync_copy(logits_hbm.at[:, pl.ds(base, per_w)], sc_v)

        def group_body(g, carry):
            tok = g * 16
            r = [jnp.full((16,), _NEG, jnp.float32) for _ in range(_K)]
            for e in range(_E):
                v = sc_v[e, pl.ds(tok, 16)]
                raw = lax.bitcast_convert_type(v, jnp.int32)
                low6 = jnp.where(raw < 0, e, (_E - 1) - e)
                x = lax.bitcast_convert_type((raw & ~(_E - 1)) | low6,
                                             jnp.float32)
                for j in range(_K):
                    hi = jnp.maximum(r[j], x)
                    x = jnp.minimum(r[j], x)
                    r[j] = hi
            mi = [lax.bitcast_convert_type(rj, jnp.int32) for rj in r]
            vals = [lax.bitcast_convert_type(m & ~(_E - 1), jnp.float32)
                    for m in mi]
            ids = [jnp.where(m < 0, m & (_E - 1), (_E - 1) - (m & (_E - 1)))
                   for m in mi]
            es = [jnp.exp(vj - vals[0]) for vj in vals]
            tot = es[0]
            for ej in es[1:]:
                tot = tot + ej
            inv = 1.0 / tot
            for j in range(_K):
                ids_v[j, pl.ds(tok, 16)] = ids[j]
                wts_v[j, pl.ds(tok, 16)] = es[j] * inv
            return carry

        lax.fori_loop(0, groups, group_body, jnp.int32(0))
        pltpu.sync_copy(ids_v, ids_hbm.at[:, pl.ds(base, per_w)])
        pltpu.sync_copy(wts_v, wts_hbm.at[:, pl.ds(base, per_w)])

    return sc_topk


@functools.partial(jax.jit, static_argnames=())
def kernel(h, W):
    b, s, d = h.shape
    n = b * s
    hf = h.reshape(n, d)
    nchunks = 4
    chunk = n // nchunks
    sc_topk = _make_sc_topk(chunk)
    ids_parts, wts_parts = [], []
    for c in range(nchunks):
        logits_t = _tc_logits_chunk(hf, W, c, chunk)
        ids_c, wts_c = sc_topk(logits_t)
        ids_parts.append(ids_c)
        wts_parts.append(wts_c)
    ids_t = jnp.concatenate(ids_parts, axis=1)
    wts_t = jnp.concatenate(wts_parts, axis=1)
    return ids_t.T, wts_t.T, jnp.float32(0.0)


# speedup vs baseline: 1.2068x; 1.2068x over previous
import functools

import jax
import jax.numpy as jnp
from jax.experimental import pallas as pl
from jax.experimental.pallas import tpu as pltpu

_E = 64
_K = 8
_NEG = -3.0e38


def _gate_block(h_ref, w_ref, ids_ref, wts_ref):
    h = h_ref[...]
    w = w_ref[...]
    logits = jax.lax.dot_general(
        w, h, (((1,), (1,)), ((), ())),
        preferred_element_type=jnp.float32)
    b = logits.shape[1]
    sub = jax.lax.broadcasted_iota(jnp.int32, (_E, b), 0)
    raw = jax.lax.bitcast_convert_type(logits, jnp.int32)
    low6 = jnp.where(raw < 0, sub, (_E - 1) - sub)
    key = jax.lax.bitcast_convert_type((raw & ~(_E - 1)) | low6, jnp.float32)
    ms = []
    for _ in range(_K):
        m = jnp.max(key, axis=0, keepdims=True)
        ms.append(m)
        key = jnp.where(key == m, _NEG, key)
    packed = jnp.concatenate(ms, axis=0)
    mi = jax.lax.bitcast_convert_type(packed, jnp.int32)
    low = mi & (_E - 1)
    ids_t = jnp.where(mi < 0, low, (_E - 1) - low)
    vals_t = jax.lax.bitcast_convert_type(mi & ~(_E - 1), jnp.float32)
    e = jnp.exp(vals_t - vals_t[:1, :])
    wts_t = e / jnp.sum(e, axis=0, keepdims=True)
    ids_ref[...] = ids_t.T
    wts_ref[...] = wts_t.T


@functools.partial(jax.jit, static_argnames=())
def kernel(h, W):
    b, s, d = h.shape
    n = b * s
    hf = h.reshape(n, d)
    blk = 4096
    grid = n // blk
    ids, wts = pl.pallas_call(
        _gate_block,
        grid=(grid,),
        in_specs=[
            pl.BlockSpec((blk, d), lambda i: (i, 0)),
            pl.BlockSpec((_E, d), lambda i: (0, 0)),
        ],
        out_specs=[
            pl.BlockSpec((blk, _K), lambda i: (i, 0)),
            pl.BlockSpec((blk, _K), lambda i: (i, 0)),
        ],
        out_shape=[
            jax.ShapeDtypeStruct((n, _K), jnp.int32),
            jax.ShapeDtypeStruct((n, _K), jnp.float32),
        ],
        compiler_params=pltpu.CompilerParams(
            dimension_semantics=("arbitrary",)),
    )(hf, W)
    return ids, wts, jnp.float32(0.0)
